# baseline (speedup 1.0000x reference)
import functools

import jax
import jax.numpy as jnp
from jax import lax
from jax.experimental import pallas as pl
from jax.experimental.pallas import tpu as pltpu
from jax.experimental.pallas import tpu_sc as plsc

f32 = jnp.float32
i32 = jnp.int32

HN = 128
GN = 50
GP = 64
CUT = 5.0
NC, NS = 2, 16
WK = NC * NS


def _ssp(x):
    return jnp.maximum(x, 0.0) + jnp.log(1.0 + jnp.exp(-jnp.abs(x))) - 0.6931471805599453


def _mesh():
    return plsc.VectorSubcoreMesh(
        core_axis_name="c", subcore_axis_name="s", num_cores=NC, num_subcores=NS)


_SC_PARAMS = pltpu.CompilerParams(needs_layout_passes=False)


def _sc_prep(EN, NV, NP, R2P):
    KC = 8
    NCH = R2P // KC
    per_w = -(-NCH // WK)
    VR = NP // 128
    VCH = -(-VR // 8)
    vper_w = -(-VCH // WK)

    @functools.partial(
        pl.kernel,
        out_type=(jax.ShapeDtypeStruct((R2P, 128), f32),
                  jax.ShapeDtypeStruct((NP, HN), f32)),
        mesh=_mesh(),
        compiler_params=_SC_PARAMS,
        scratch_types=[
            pltpu.VMEM((-(-(NV * 3) // 1024) * 8, 128), f32),
            pltpu.VMEM((KC, 128), i32),
            pltpu.VMEM((KC, 128), i32),
            pltpu.VMEM((KC, 128), f32),
            pltpu.VMEM((8, 128), i32),
            pltpu.VMEM((128, HN), f32),
        ],
    )
    def k(pos_hbm, row_hbm, col_hbm, z_hbm, initv_hbm,
          d2_out, v0_out, posv, rowb, colb, d2b, zb, vrows):
        c = lax.axis_index("c")
        s = lax.axis_index("s")
        wid = s * NC + c
        pltpu.sync_copy(pos_hbm, posv)

        def chunk(i, carry):
            cid = i * WK + wid

            @pl.when(cid < NCH)
            def _():
                pltpu.sync_copy(row_hbm.at[pl.ds(cid * KC, KC)], rowb)
                pltpu.sync_copy(col_hbm.at[pl.ds(cid * KC, KC)], colb)
                def g(fi):
                    return plsc.load_gather(
                        posv, [lax.shift_right_logical(fi, 7),
                               lax.bitwise_and(fi, 127)])

                for r in range(KC):
                    for l in range(8):
                        ri = rowb[r, pl.ds(l * 16, 16)] * 3
                        ci = colb[r, pl.ds(l * 16, 16)] * 3
                        dx = g(ri) - g(ci)
                        dy = g(ri + 1) - g(ci + 1)
                        dz = g(ri + 2) - g(ci + 2)
                        d2b[r, pl.ds(l * 16, 16)] = dx * dx + dy * dy + dz * dz
                pltpu.sync_copy(d2b, d2_out.at[pl.ds(cid * KC, KC)])
            return carry

        lax.fori_loop(0, per_w, chunk, 0)

        def vchunk(i, carry):
            cid = i * WK + wid

            @pl.when(cid < VCH)
            def _():
                pltpu.sync_copy(z_hbm.at[pl.ds(cid * 8, 8)], zb)
                for q in range(8):
                    r = cid * 8 + q

                    @pl.when(r < VR)
                    def _():
                        pltpu.sync_copy(initv_hbm.at[zb.at[q]], vrows)
                        pltpu.sync_copy(vrows, v0_out.at[pl.ds(r * 128, 128)])
            return carry

        lax.fori_loop(0, vper_w, vchunk, 0)

    return k


def _sc_edge(EN, NP):
    KC = 8
    CE = 128
    NCH = -(-EN // (KC * 128))
    per_w = -(-NCH // WK)
    TPS = NP // NS

    @functools.partial(
        pl.kernel,
        out_type=jax.ShapeDtypeStruct((NC, NP, HN), f32),
        mesh=_mesh(),
        compiler_params=_SC_PARAMS,
        scratch_types=[
            pltpu.VMEM((KC, 128), i32),
            pltpu.VMEM((KC, 128), i32),
            pltpu.VMEM((CE, HN), f32),
            pltpu.VMEM((CE, HN), f32),
            pltpu.VMEM_SHARED((NP, HN), f32)
        ],
    )
    def k(vl_hbm, w_hbm, row_hbm, col_hbm, zeros_hbm,
          out_hbm, rowb, colb, vlr, wch, acc):
        c = lax.axis_index("c")
        s = lax.axis_index("s")
        wid = s * NC + c
        pltpu.sync_copy(zeros_hbm, acc.at[pl.ds(s * TPS, TPS)])
        plsc.subcore_barrier()

        def chunk(i, carry):
            cid = i * WK + wid

            @pl.when(cid < NCH)
            def _():
                pltpu.sync_copy(row_hbm.at[pl.ds(cid * KC, KC)], rowb)
                pltpu.sync_copy(col_hbm.at[pl.ds(cid * KC, KC)], colb)
                base = cid * KC * 128
                for q in range(KC):
                    @pl.when(base + (q + 1) * CE <= EN)
                    def _():
                        pltpu.sync_copy(vl_hbm.at[rowb.at[q]], vlr)
                        pltpu.sync_copy(w_hbm.at[pl.ds(base + q * CE, CE)], wch)

                        @plsc.parallel_loop(0, CE, 1, unroll=4)
                        def _(r):
                            for l in range(8):
                                wch[r, pl.ds(l * 16, 16)] = (
                                    wch[r, pl.ds(l * 16, 16)]
                                    * vlr[r, pl.ds(l * 16, 16)])
                        pltpu.sync_copy(wch, acc.at[colb.at[q]], add=True)
            return carry

        lax.fori_loop(0, per_w, chunk, 0)
        plsc.subcore_barrier()
        pltpu.sync_copy(acc.at[pl.ds(s * TPS, TPS)],
                        out_hbm.at[c, pl.ds(s * TPS, TPS)])

    return k


def _sc_readout(NP, NSEG):
    VR = NP // 128
    VCH = -(-VR // 8)
    per_w = -(-VCH // NS)
    ZR = 128 // NS

    @functools.partial(
        pl.kernel,
        out_type=jax.ShapeDtypeStruct((NSEG, HN), f32),
        mesh=_mesh(),
        compiler_params=_SC_PARAMS,
        scratch_types=[
            pltpu.VMEM((8, 128), i32),
            pltpu.VMEM((128, HN), f32),
            pltpu.VMEM_SHARED((128, HN), f32),
        ],
    )
    def k(h_hbm, batch_hbm, zeros_hbm, u_out, bb, hb, acc):
        c = lax.axis_index("c")
        s = lax.axis_index("s")

        @pl.when(c == 0)
        def _():
            pltpu.sync_copy(zeros_hbm.at[pl.ds(0, ZR)], acc.at[pl.ds(s * ZR, ZR)])
        plsc.subcore_barrier()

        def chunk(i, carry):
            cid = i * NS + s

            @pl.when((c == 0) & (cid < VCH))
            def _():
                pltpu.sync_copy(batch_hbm.at[pl.ds(cid * 8, 8)], bb)
                for q in range(8):
                    r = cid * 8 + q

                    @pl.when(r < VR)
                    def _():
                        pltpu.sync_copy(h_hbm.at[pl.ds(r * 128, 128)], hb)
                        pltpu.sync_copy(hb, acc.at[bb.at[q]], add=True)
            return carry

        lax.fori_loop(0, per_w, chunk, 0)
        plsc.subcore_barrier()

        @pl.when((c == 0) & (s < NSEG // 8))
        def _():
            pltpu.sync_copy(acc.at[pl.ds(s * 8, 8)], u_out.at[pl.ds(s * 8, 8)])

    return k


def _tc_filter(R2P):
    KJ = 16
    BE = KJ * 128
    EP = R2P * 128
    grid = R2P // KJ
    step = CUT / (GN - 1)
    coeff = -0.5 / (step * step)

    def body(d2_ref, w1_ref, b1_ref, w2_ref, b2_ref, w0_ref, wl1_ref, descr):
        for j in range(KJ):
            d2r = d2_ref[j:j + 1, :]
            dist = jnp.sqrt(d2r)
            rid = lax.broadcasted_iota(i32, (GP, 128), 0)
            offs = rid.astype(f32) * step
            det = jnp.exp(coeff * (dist - offs) ** 2)
            cenv = 0.5 * (jnp.cos(dist * (jnp.pi / CUT)) + 1.0)
            m = jnp.where(rid < GN, det,
                          jnp.where(rid == GN, jnp.broadcast_to(cenv, (GP, 128)),
                                    0.0))
            descr[pl.ds(j * 128, 128), :] = m.T
        de = descr[...]
        ccol = de[:, GN:GN + 1]
        for l, out_ref in ((0, w0_ref), (1, wl1_ref)):
            h1 = _ssp(jnp.dot(de, w1_ref[l], preferred_element_type=f32)
                      + b1_ref[l])
            out_ref[...] = (jnp.dot(h1, w2_ref[l], preferred_element_type=f32)
                            + b2_ref[l]) * ccol

    wspec = lambda shp: pl.BlockSpec(shp, lambda i: (0,) * len(shp))
    return pl.pallas_call(
        body,
        grid=(grid,),
        in_specs=[
            pl.BlockSpec((KJ, 128), lambda i: (i, 0)),
            wspec((2, GP, HN)), wspec((2, 1, HN)),
            wspec((2, HN, HN)), wspec((2, 1, HN)),
        ],
        out_specs=[
            pl.BlockSpec((BE, HN), lambda i: (i, 0)),
            pl.BlockSpec((BE, HN), lambda i: (i, 0)),
        ],
        out_shape=[jax.ShapeDtypeStruct((EP, HN), f32),
                   jax.ShapeDtypeStruct((EP, HN), f32)],
        scratch_shapes=[pltpu.VMEM((BE, GP), f32)],
    )


def _tc_matmul(NP, BN):
    def body(v_ref, w_ref, o_ref):
        o_ref[...] = jnp.dot(v_ref[...], w_ref[...], preferred_element_type=f32)

    return pl.pallas_call(
        body,
        grid=(NP // BN,),
        in_specs=[pl.BlockSpec((BN, HN), lambda i: (i, 0)),
                  pl.BlockSpec((HN, HN), lambda i: (0, 0))],
        out_specs=pl.BlockSpec((BN, HN), lambda i: (i, 0)),
        out_shape=jax.ShapeDtypeStruct((NP, HN), f32),
    )


def _tc_node0(NP, BN):
    def body(p_ref, v_ref, w1_ref, b1_ref, w2_ref, b2_ref, el_ref,
             vout_ref, vlout_ref):
        po = p_ref[0] + p_ref[1]
        o = _ssp(jnp.dot(po, w1_ref[...], preferred_element_type=f32) + b1_ref[...])
        o = jnp.dot(o, w2_ref[...], preferred_element_type=f32) + b2_ref[...]
        vn = v_ref[...] + o
        vout_ref[...] = vn
        vlout_ref[...] = jnp.dot(vn, el_ref[...], preferred_element_type=f32)

    return pl.pallas_call(
        body,
        grid=(NP // BN,),
        in_specs=[
            pl.BlockSpec((NC, BN, HN), lambda i: (0, i, 0)),
            pl.BlockSpec((BN, HN), lambda i: (i, 0)),
            pl.BlockSpec((HN, HN), lambda i: (0, 0)),
            pl.BlockSpec((1, HN), lambda i: (0, 0)),
            pl.BlockSpec((HN, HN), lambda i: (0, 0)),
            pl.BlockSpec((1, HN), lambda i: (0, 0)),
            pl.BlockSpec((HN, HN), lambda i: (0, 0)),
        ],
        out_specs=[pl.BlockSpec((BN, HN), lambda i: (i, 0)),
                   pl.BlockSpec((BN, HN), lambda i: (i, 0))],
        out_shape=[jax.ShapeDtypeStruct((NP, HN), f32),
                   jax.ShapeDtypeStruct((NP, HN), f32)],
    )


def _tc_node1(NP, BN, HH):
    def body(p_ref, v_ref, w1_ref, b1_ref, w2_ref, b2_ref,
             u1_ref, ub1_ref, u2_ref, ub2_ref, h_ref):
        po = p_ref[0] + p_ref[1]
        o = _ssp(jnp.dot(po, w1_ref[...], preferred_element_type=f32) + b1_ref[...])
        o = jnp.dot(o, w2_ref[...], preferred_element_type=f32) + b2_ref[...]
        vn = v_ref[...] + o
        h1 = _ssp(jnp.dot(vn, u1_ref[...], preferred_element_type=f32) + ub1_ref[...])
        h = jnp.sum(h1 * u2_ref[...], axis=1, keepdims=True) + ub2_ref[...]
        h_ref[...] = jnp.broadcast_to(h, (BN, HN))

    return pl.pallas_call(
        body,
        grid=(NP // BN,),
        in_specs=[
            pl.BlockSpec((NC, BN, HN), lambda i: (0, i, 0)),
            pl.BlockSpec((BN, HN), lambda i: (i, 0)),
            pl.BlockSpec((HN, HN), lambda i: (0, 0)),
            pl.BlockSpec((1, HN), lambda i: (0, 0)),
            pl.BlockSpec((HN, HN), lambda i: (0, 0)),
            pl.BlockSpec((1, HN), lambda i: (0, 0)),
            pl.BlockSpec((HN, HH), lambda i: (0, 0)),
            pl.BlockSpec((1, HH), lambda i: (0, 0)),
            pl.BlockSpec((1, HH), lambda i: (0, 0)),
            pl.BlockSpec((1, 1), lambda i: (0, 0)),
        ],
        out_specs=pl.BlockSpec((BN, HN), lambda i: (i, 0)),
        out_shape=jax.ShapeDtypeStruct((NP, HN), f32),
    )


def kernel(z, pos, batch, edge_index, init_v,
           e_mlp_w1, e_mlp_b1, e_mlp_w2, e_mlp_b2, e_lin_w,
           v_lin1_w, v_lin1_b, v_lin2_w, v_lin2_b,
           u_lin1_w, u_lin1_b, u_lin2_w, u_lin2_b):
    NV = pos.shape[0]
    EN = edge_index.shape[1]
    NP = -(-NV // 128) * 128
    R2 = EN // 128
    R2P = -(-R2 // 16) * 16
    NSEG = 64
    HH = u_lin1_w.shape[1]
    BN = NP // 8

    row2d = jnp.zeros((R2P, 128), i32).at[:R2].set(
        edge_index[0].astype(i32).reshape(R2, 128))
    col2d = jnp.zeros((R2P, 128), i32).at[:R2].set(
        edge_index[1].astype(i32).reshape(R2, 128))
    PR = -(-(NV * 3) // 1024) * 8
    pos_flat = jnp.zeros((PR * 128,), f32).at[:NV * 3].set(
        pos.astype(f32).reshape(NV * 3)).reshape(PR, 128)
    NPZ = (-(-(NP // 128) // 8) * 8) * 128
    z_pad = jnp.zeros((NPZ,), i32).at[:NV].set(
        z.astype(i32)).reshape(NPZ // 128, 128)
    batch_pad = jnp.full((NPZ,), NSEG, i32).at[:NV].set(
        batch.astype(i32)).reshape(NPZ // 128, 128)
    zeros_acc = jnp.zeros((NP // NS, HN), f32)

    w1p = jnp.zeros((2, GP, HN), f32).at[:, :GN, :].set(e_mlp_w1)
    b1p = e_mlp_b1.reshape(2, 1, HN)
    b2p = e_mlp_b2.reshape(2, 1, HN)

    d2, v0 = _sc_prep(EN, NV, NP, R2P)(pos_flat, row2d, col2d, z_pad,
                                       init_v.astype(f32))
    w_both = _tc_filter(R2P)(d2, w1p, b1p, e_mlp_w2, b2p)
    edge_k = _sc_edge(EN, NP)

    vl = _tc_matmul(NP, BN)(v0, e_lin_w[0])
    p0 = edge_k(vl, w_both[0], row2d, col2d, zeros_acc)
    v1, vl1 = _tc_node0(NP, BN)(p0, v0, v_lin1_w[0], v_lin1_b[0].reshape(1, HN),
                                v_lin2_w[0], v_lin2_b[0].reshape(1, HN),
                                e_lin_w[1])
    p1 = edge_k(vl1, w_both[1], row2d, col2d, zeros_acc)
    h = _tc_node1(NP, BN, HH)(p1, v1, v_lin1_w[1], v_lin1_b[1].reshape(1, HN),
                              v_lin2_w[1], v_lin2_b[1].reshape(1, HN),
                              u_lin1_w, u_lin1_b.reshape(1, HH),
                              u_lin2_w.reshape(1, HH), u_lin2_b.reshape(1, 1))
    u = _sc_readout(NP, NSEG)(h, batch_pad, zeros_acc)
    return u[:, 0:1]


# speedup vs baseline: 1.0731x; 1.0731x over previous
import functools

import jax
import jax.numpy as jnp
from jax import lax
from jax.experimental import pallas as pl
from jax.experimental.pallas import tpu as pltpu
from jax.experimental.pallas import tpu_sc as plsc

f32 = jnp.float32
i32 = jnp.int32

HN = 128
GN = 50
GP = 64
CUT = 5.0
NC, NS = 2, 16
WK = NC * NS


def _ssp(x):
    return jnp.maximum(x, 0.0) + jnp.log(1.0 + jnp.exp(-jnp.abs(x))) - 0.6931471805599453


def _mesh():
    return plsc.VectorSubcoreMesh(
        core_axis_name="c", subcore_axis_name="s", num_cores=NC, num_subcores=NS)


_SC_PARAMS = pltpu.CompilerParams(needs_layout_passes=False)


def _sc_prep(EN, NV, NP, R2P):
    KC = 8
    NCH = R2P // KC
    per_w = -(-NCH // WK)
    VR = NP // 128
    VCH = -(-VR // 8)
    vper_w = -(-VCH // WK)

    @functools.partial(
        pl.kernel,
        out_type=(jax.ShapeDtypeStruct((R2P, 128), f32),
                  jax.ShapeDtypeStruct((NP, HN), f32)),
        mesh=_mesh(),
        compiler_params=_SC_PARAMS,
        scratch_types=[
            pltpu.VMEM((-(-(NV * 3) // 1024) * 8, 128), f32),
            pltpu.VMEM((KC, 128), i32),
            pltpu.VMEM((KC, 128), i32),
            pltpu.VMEM((KC, 128), f32),
            pltpu.VMEM((8, 128), i32),
            pltpu.VMEM((128, HN), f32),
        ],
    )
    def k(pos_hbm, row_hbm, col_hbm, z_hbm, initv_hbm,
          d2_out, v0_out, posv, rowb, colb, d2b, zb, vrows):
        c = lax.axis_index("c")
        s = lax.axis_index("s")
        wid = s * NC + c
        pltpu.sync_copy(pos_hbm, posv)

        def chunk(i, carry):
            cid = i * WK + wid

            @pl.when(cid < NCH)
            def _():
                pltpu.sync_copy(row_hbm.at[pl.ds(cid * KC, KC)], rowb)
                pltpu.sync_copy(col_hbm.at[pl.ds(cid * KC, KC)], colb)
                def g(fi):
                    return plsc.load_gather(
                        posv, [lax.shift_right_logical(fi, 7),
                               lax.bitwise_and(fi, 127)])

                for r in range(KC):
                    for l in range(8):
                        ri = rowb[r, pl.ds(l * 16, 16)] * 3
                        ci = colb[r, pl.ds(l * 16, 16)] * 3
                        dx = g(ri) - g(ci)
                        dy = g(ri + 1) - g(ci + 1)
                        dz = g(ri + 2) - g(ci + 2)
                        d2b[r, pl.ds(l * 16, 16)] = dx * dx + dy * dy + dz * dz
                pltpu.sync_copy(d2b, d2_out.at[pl.ds(cid * KC, KC)])
            return carry

        lax.fori_loop(0, per_w, chunk, 0)

        def vchunk(i, carry):
            cid = i * WK + wid

            @pl.when(cid < VCH)
            def _():
                pltpu.sync_copy(z_hbm.at[pl.ds(cid * 8, 8)], zb)
                for q in range(8):
                    r = cid * 8 + q

                    @pl.when(r < VR)
                    def _():
                        pltpu.sync_copy(initv_hbm.at[zb.at[q]], vrows)
                        pltpu.sync_copy(vrows, v0_out.at[pl.ds(r * 128, 128)])
            return carry

        lax.fori_loop(0, vper_w, vchunk, 0)

    return k


def _sc_edge_gm(EN, NP):
    KC = 8
    CE = 128
    NCH = -(-EN // (KC * 128))
    per_w = -(-NCH // WK)

    @functools.partial(
        pl.kernel,
        out_type=jax.ShapeDtypeStruct((EN, HN), f32),
        mesh=_mesh(),
        compiler_params=_SC_PARAMS,
        scratch_types=[
            pltpu.VMEM((KC, 128), i32),
            pltpu.VMEM((2, CE, HN), f32),
            pltpu.VMEM((2, CE, HN), f32),
            pltpu.VMEM((2, CE, HN), f32),
            pltpu.SemaphoreType.DMA,
            pltpu.SemaphoreType.DMA,
            pltpu.SemaphoreType.DMA,
            pltpu.SemaphoreType.DMA,
            pltpu.SemaphoreType.DMA,
            pltpu.SemaphoreType.DMA,
        ],
    )
    def k(vl_hbm, w_hbm, row_hbm, e_out, rowb, vlr, wch, ech,
          gs0, gs1, ws0, ws1, es0, es1):
        c = lax.axis_index("c")
        s = lax.axis_index("s")
        wid = s * NC + c
        gs = (gs0, gs1)
        ws = (ws0, ws1)
        es = (es0, es1)

        def chunk(i, carry):
            cid = i * WK + wid

            @pl.when(cid < NCH)
            def _():
                pltpu.sync_copy(row_hbm.at[pl.ds(cid * KC, KC)], rowb)
                base = cid * KC * 128

                def valid(q):
                    return base + (q + 1) * CE <= EN

                def start_in(q):
                    b = q & 1
                    pltpu.async_copy(vl_hbm.at[rowb.at[q]], vlr.at[b], gs[b])
                    pltpu.async_copy(w_hbm.at[pl.ds(base + q * CE, CE)],
                                     wch.at[b], ws[b])

                for q in range(KC):
                    b = q & 1

                    @pl.when(valid(q))
                    def _():
                        if q < 2:
                            start_in(q)
                        pltpu.make_async_copy(
                            vl_hbm.at[rowb.at[q]], vlr.at[b], gs[b]).wait()
                        pltpu.make_async_copy(
                            w_hbm.at[pl.ds(base + q * CE, CE)],
                            wch.at[b], ws[b]).wait()
                        if q >= 2:
                            pltpu.make_async_copy(
                                ech.at[b],
                                e_out.at[pl.ds(base + (q - 2) * CE, CE)],
                                es[b]).wait()

                        @plsc.parallel_loop(0, CE, 1, unroll=4)
                        def _(r):
                            for l in range(8):
                                ech[b, r, pl.ds(l * 16, 16)] = (
                                    wch[b, r, pl.ds(l * 16, 16)]
                                    * vlr[b, r, pl.ds(l * 16, 16)])

                        pltpu.async_copy(
                            ech.at[b], e_out.at[pl.ds(base + q * CE, CE)],
                            es[b])
                        if q + 2 < KC:
                            @pl.when(valid(q + 2))
                            def _():
                                start_in(q + 2)
                for q in range(KC):
                    cond = valid(q)
                    if q + 2 < KC:
                        cond = cond & jnp.logical_not(valid(q + 2))

                    @pl.when(cond)
                    def _():
                        pltpu.make_async_copy(
                            ech.at[q & 1],
                            e_out.at[pl.ds(base + q * CE, CE)],
                            es[q & 1]).wait()
            return carry

        lax.fori_loop(0, per_w, chunk, 0)

    return k


def _sc_edge_scat(EN, NP):
    KC = 8
    CE = 128
    NCH = -(-EN // (KC * 128))
    per_w = -(-NCH // WK)
    TPS = NP // NS

    @functools.partial(
        pl.kernel,
        out_type=jax.ShapeDtypeStruct((NC, NP, HN), f32),
        mesh=_mesh(),
        compiler_params=_SC_PARAMS,
        scratch_types=[
            pltpu.VMEM((KC, 128), i32),
            pltpu.VMEM((2, CE, HN), f32),
            pltpu.VMEM_SHARED((NP, HN), f32),
            pltpu.SemaphoreType.DMA,
            pltpu.SemaphoreType.DMA,
        ],
    )
    def k(e_hbm, col_hbm, zeros_hbm, out_hbm, colb, ech, acc, is0, is1):
        c = lax.axis_index("c")
        s = lax.axis_index("s")
        wid = s * NC + c
        isem = (is0, is1)
        pltpu.sync_copy(zeros_hbm, acc.at[pl.ds(s * TPS, TPS)])
        plsc.subcore_barrier()

        def chunk(i, carry):
            cid = i * WK + wid

            @pl.when(cid < NCH)
            def _():
                pltpu.sync_copy(col_hbm.at[pl.ds(cid * KC, KC)], colb)
                base = cid * KC * 128

                def valid(q):
                    return base + (q + 1) * CE <= EN

                for q in range(KC):
                    b = q & 1

                    @pl.when(valid(q))
                    def _():
                        if q < 2:
                            pltpu.async_copy(
                                e_hbm.at[pl.ds(base + q * CE, CE)],
                                ech.at[b], isem[b])
                        pltpu.make_async_copy(
                            e_hbm.at[pl.ds(base + q * CE, CE)],
                            ech.at[b], isem[b]).wait()
                        pltpu.sync_copy(ech.at[b], acc.at[colb.at[q]],
                                        add=True)
                        if q + 2 < KC:
                            @pl.when(valid(q + 2))
                            def _():
                                pltpu.async_copy(
                                    e_hbm.at[pl.ds(base + (q + 2) * CE, CE)],
                                    ech.at[b], isem[b])
            return carry

        lax.fori_loop(0, per_w, chunk, 0)
        plsc.subcore_barrier()
        pltpu.sync_copy(acc.at[pl.ds(s * TPS, TPS)],
                        out_hbm.at[c, pl.ds(s * TPS, TPS)])

    return k


def _sc_readout(NP, NSEG):
    VR = NP // 128
    VCH = -(-VR // 8)
    per_w = -(-VCH // NS)
    ZR = 128 // NS

    @functools.partial(
        pl.kernel,
        out_type=jax.ShapeDtypeStruct((NSEG, HN), f32),
        mesh=_mesh(),
        compiler_params=_SC_PARAMS,
        scratch_types=[
            pltpu.VMEM((8, 128), i32),
            pltpu.VMEM((128, HN), f32),
            pltpu.VMEM_SHARED((128, HN), f32),
        ],
    )
    def k(h_hbm, batch_hbm, zeros_hbm, u_out, bb, hb, acc):
        c = lax.axis_index("c")
        s = lax.axis_index("s")

        @pl.when(c == 0)
        def _():
            pltpu.sync_copy(zeros_hbm.at[pl.ds(0, ZR)], acc.at[pl.ds(s * ZR, ZR)])
        plsc.subcore_barrier()

        def chunk(i, carry):
            cid = i * NS + s

            @pl.when((c == 0) & (cid < VCH))
            def _():
                pltpu.sync_copy(batch_hbm.at[pl.ds(cid * 8, 8)], bb)
                for q in range(8):
                    r = cid * 8 + q

                    @pl.when(r < VR)
                    def _():
                        pltpu.sync_copy(h_hbm.at[pl.ds(r * 128, 128)], hb)
                        pltpu.sync_copy(hb, acc.at[bb.at[q]], add=True)
            return carry

        lax.fori_loop(0, per_w, chunk, 0)
        plsc.subcore_barrier()

        @pl.when((c == 0) & (s < NSEG // 8))
        def _():
            pltpu.sync_copy(acc.at[pl.ds(s * 8, 8)], u_out.at[pl.ds(s * 8, 8)])

    return k


def _tc_filter(R2P):
    KJ = 16
    BE = KJ * 128
    EP = R2P * 128
    grid = R2P // KJ
    step = CUT / (GN - 1)
    coeff = -0.5 / (step * step)

    def body(d2_ref, w1_ref, b1_ref, w2_ref, b2_ref, w0_ref, wl1_ref, descr):
        for j in range(KJ):
            d2r = d2_ref[j:j + 1, :]
            dist = jnp.sqrt(d2r)
            rid = lax.broadcasted_iota(i32, (GP, 128), 0)
            offs = rid.astype(f32) * step
            det = jnp.exp(coeff * (dist - offs) ** 2)
            cenv = 0.5 * (jnp.cos(dist * (jnp.pi / CUT)) + 1.0)
            m = jnp.where(rid < GN, det,
                          jnp.where(rid == GN, jnp.broadcast_to(cenv, (GP, 128)),
                                    0.0))
            descr[pl.ds(j * 128, 128), :] = m.T
        de = descr[...]
        ccol = de[:, GN:GN + 1]
        for l, out_ref in ((0, w0_ref), (1, wl1_ref)):
            h1 = _ssp(jnp.dot(de, w1_ref[l], preferred_element_type=f32)
                      + b1_ref[l])
            out_ref[...] = (jnp.dot(h1, w2_ref[l], preferred_element_type=f32)
                            + b2_ref[l]) * ccol

    wspec = lambda shp: pl.BlockSpec(shp, lambda i: (0,) * len(shp))
    return pl.pallas_call(
        body,
        grid=(grid,),
        in_specs=[
            pl.BlockSpec((KJ, 128), lambda i: (i, 0)),
            wspec((2, GP, HN)), wspec((2, 1, HN)),
            wspec((2, HN, HN)), wspec((2, 1, HN)),
        ],
        out_specs=[
            pl.BlockSpec((BE, HN), lambda i: (i, 0)),
            pl.BlockSpec((BE, HN), lambda i: (i, 0)),
        ],
        out_shape=[jax.ShapeDtypeStruct((EP, HN), f32),
                   jax.ShapeDtypeStruct((EP, HN), f32)],
        scratch_shapes=[pltpu.VMEM((BE, GP), f32)],
    )


def _tc_matmul(NP, BN):
    def body(v_ref, w_ref, o_ref):
        o_ref[...] = jnp.dot(v_ref[...], w_ref[...], preferred_element_type=f32)

    return pl.pallas_call(
        body,
        grid=(NP // BN,),
        in_specs=[pl.BlockSpec((BN, HN), lambda i: (i, 0)),
                  pl.BlockSpec((HN, HN), lambda i: (0, 0))],
        out_specs=pl.BlockSpec((BN, HN), lambda i: (i, 0)),
        out_shape=jax.ShapeDtypeStruct((NP, HN), f32),
    )


def _tc_node0(NP, BN):
    def body(p_ref, v_ref, w1_ref, b1_ref, w2_ref, b2_ref, el_ref,
             vout_ref, vlout_ref):
        po = p_ref[0] + p_ref[1]
        o = _ssp(jnp.dot(po, w1_ref[...], preferred_element_type=f32) + b1_ref[...])
        o = jnp.dot(o, w2_ref[...], preferred_element_type=f32) + b2_ref[...]
        vn = v_ref[...] + o
        vout_ref[...] = vn
        vlout_ref[...] = jnp.dot(vn, el_ref[...], preferred_element_type=f32)

    return pl.pallas_call(
        body,
        grid=(NP // BN,),
        in_specs=[
            pl.BlockSpec((NC, BN, HN), lambda i: (0, i, 0)),
            pl.BlockSpec((BN, HN), lambda i: (i, 0)),
            pl.BlockSpec((HN, HN), lambda i: (0, 0)),
            pl.BlockSpec((1, HN), lambda i: (0, 0)),
            pl.BlockSpec((HN, HN), lambda i: (0, 0)),
            pl.BlockSpec((1, HN), lambda i: (0, 0)),
            pl.BlockSpec((HN, HN), lambda i: (0, 0)),
        ],
        out_specs=[pl.BlockSpec((BN, HN), lambda i: (i, 0)),
                   pl.BlockSpec((BN, HN), lambda i: (i, 0))],
        out_shape=[jax.ShapeDtypeStruct((NP, HN), f32),
                   jax.ShapeDtypeStruct((NP, HN), f32)],
    )


def _tc_node1(NP, BN, HH):
    def body(p_ref, v_ref, w1_ref, b1_ref, w2_ref, b2_ref,
             u1_ref, ub1_ref, u2_ref, ub2_ref, h_ref):
        po = p_ref[0] + p_ref[1]
        o = _ssp(jnp.dot(po, w1_ref[...], preferred_element_type=f32) + b1_ref[...])
        o = jnp.dot(o, w2_ref[...], preferred_element_type=f32) + b2_ref[...]
        vn = v_ref[...] + o
        h1 = _ssp(jnp.dot(vn, u1_ref[...], preferred_element_type=f32) + ub1_ref[...])
        h = jnp.sum(h1 * u2_ref[...], axis=1, keepdims=True) + ub2_ref[...]
        h_ref[...] = jnp.broadcast_to(h, (BN, HN))

    return pl.pallas_call(
        body,
        grid=(NP // BN,),
        in_specs=[
            pl.BlockSpec((NC, BN, HN), lambda i: (0, i, 0)),
            pl.BlockSpec((BN, HN), lambda i: (i, 0)),
            pl.BlockSpec((HN, HN), lambda i: (0, 0)),
            pl.BlockSpec((1, HN), lambda i: (0, 0)),
            pl.BlockSpec((HN, HN), lambda i: (0, 0)),
            pl.BlockSpec((1, HN), lambda i: (0, 0)),
            pl.BlockSpec((HN, HH), lambda i: (0, 0)),
            pl.BlockSpec((1, HH), lambda i: (0, 0)),
            pl.BlockSpec((1, HH), lambda i: (0, 0)),
            pl.BlockSpec((1, 1), lambda i: (0, 0)),
        ],
        out_specs=pl.BlockSpec((BN, HN), lambda i: (i, 0)),
        out_shape=jax.ShapeDtypeStruct((NP, HN), f32),
    )


def kernel(z, pos, batch, edge_index, init_v,
           e_mlp_w1, e_mlp_b1, e_mlp_w2, e_mlp_b2, e_lin_w,
           v_lin1_w, v_lin1_b, v_lin2_w, v_lin2_b,
           u_lin1_w, u_lin1_b, u_lin2_w, u_lin2_b):
    NV = pos.shape[0]
    EN = edge_index.shape[1]
    NP = -(-NV // 128) * 128
    R2 = EN // 128
    R2P = -(-R2 // 16) * 16
    NSEG = 64
    HH = u_lin1_w.shape[1]
    BN = NP // 8

    row2d = jnp.zeros((R2P, 128), i32).at[:R2].set(
        edge_index[0].astype(i32).reshape(R2, 128))
    col2d = jnp.zeros((R2P, 128), i32).at[:R2].set(
        edge_index[1].astype(i32).reshape(R2, 128))
    PR = -(-(NV * 3) // 1024) * 8
    pos_flat = jnp.zeros((PR * 128,), f32).at[:NV * 3].set(
        pos.astype(f32).reshape(NV * 3)).reshape(PR, 128)
    NPZ = (-(-(NP // 128) // 8) * 8) * 128
    z_pad = jnp.zeros((NPZ,), i32).at[:NV].set(
        z.astype(i32)).reshape(NPZ // 128, 128)
    batch_pad = jnp.full((NPZ,), NSEG, i32).at[:NV].set(
        batch.astype(i32)).reshape(NPZ // 128, 128)
    zeros_acc = jnp.zeros((NP // NS, HN), f32)

    w1p = jnp.zeros((2, GP, HN), f32).at[:, :GN, :].set(e_mlp_w1)
    b1p = e_mlp_b1.reshape(2, 1, HN)
    b2p = e_mlp_b2.reshape(2, 1, HN)

    d2, v0 = _sc_prep(EN, NV, NP, R2P)(pos_flat, row2d, col2d, z_pad,
                                       init_v.astype(f32))
    w_both = _tc_filter(R2P)(d2, w1p, b1p, e_mlp_w2, b2p)
    gm_k = _sc_edge_gm(EN, NP)
    scat_k = _sc_edge_scat(EN, NP)

    vl = _tc_matmul(NP, BN)(v0, e_lin_w[0])
    e0 = gm_k(vl, w_both[0], row2d)
    p0 = scat_k(e0, col2d, zeros_acc)
    v1, vl1 = _tc_node0(NP, BN)(p0, v0, v_lin1_w[0], v_lin1_b[0].reshape(1, HN),
                                v_lin2_w[0], v_lin2_b[0].reshape(1, HN),
                                e_lin_w[1])
    e1 = gm_k(vl1, w_both[1], row2d)
    p1 = scat_k(e1, col2d, zeros_acc)
    h = _tc_node1(NP, BN, HH)(p1, v1, v_lin1_w[1], v_lin1_b[1].reshape(1, HN),
                              v_lin2_w[1], v_lin2_b[1].reshape(1, HN),
                              u_lin1_w, u_lin1_b.reshape(1, HH),
                              u_lin2_w.reshape(1, HH), u_lin2_b.reshape(1, 1))
    u = _sc_readout(NP, NSEG)(h, batch_pad, zeros_acc)
    return u[:, 0:1]
